# baseline (speedup 1.0000x reference)
import jax
import jax.numpy as jnp
from jax.experimental import pallas as pl
from jax.experimental.pallas import tpu as pltpu

CAP = 65536
X_DIM = 128
Y0, Y1 = 32, 32
Y_FLAT = Y0 * Y1

CAM_SR = 512
NB_C = 8
CAM_STRIPES = CAP // CAM_SR
CAM_GROUPS = CAM_STRIPES // NB_C

ST_SR = 4096
NB_S = 4
ST_STRIPES = CAP // ST_SR
ST_GROUPS = ST_STRIPES // NB_S

ITER_R = CAP // 128


def _push_body(pos_ref, cnt_ref, state_v, cam_v, sb_any, cb_any, it_any,
               sb_out, cb_out, it_out,
               cam_buf, st_buf, it_buf,
               sem_ci, sem_co, sem_si, sem_so, sem_it, sem_rows):
    pos = pos_ref[0]
    cnt = cnt_ref[0]

    def cam_in(g, k, p):
        s = g * NB_C + k
        return pltpu.make_async_copy(
            cb_any.at[pl.ds(s * CAM_SR, CAM_SR)], cam_buf.at[p, k],
            sem_ci.at[p, k])

    def cam_out(g, k, p):
        s = g * NB_C + k
        return pltpu.make_async_copy(
            cam_buf.at[p, k], cb_out.at[pl.ds(s * CAM_SR, CAM_SR)],
            sem_co.at[p, k])

    def st_in(g, k, p):
        s = g * NB_S + k
        return pltpu.make_async_copy(
            sb_any.at[pl.ds(s * ST_SR, ST_SR)], st_buf.at[p, k],
            sem_si.at[p, k])

    def st_out(g, k, p):
        s = g * NB_S + k
        return pltpu.make_async_copy(
            st_buf.at[p, k], sb_out.at[pl.ds(s * ST_SR, ST_SR)],
            sem_so.at[p, k])

    it_in = pltpu.make_async_copy(it_any, it_buf, sem_it)
    it_wr = pltpu.make_async_copy(it_buf, it_out, sem_it)

    it_in.start()
    for k in range(NB_C):
        cam_in(0, k, 0).start()
    for k in range(NB_S):
        st_in(0, k, 0).start()

    for g in range(CAM_GROUPS):
        p = g % 2
        for k in range(NB_C):
            cam_in(g, k, p).wait()
            cam_out(g, k, p).start()
        if g + 1 < CAM_GROUPS:
            for k in range(NB_C):
                cam_in(g + 1, k, 1 - p).start()
        for k in range(NB_C):
            cam_out(g, k, p).wait()

    row_c = pltpu.make_async_copy(cam_v, cb_out.at[pl.ds(pos, 1)],
                                  sem_rows.at[1])
    row_c.start()

    for g in range(ST_GROUPS):
        p = g % 2
        for k in range(NB_S):
            st_in(g, k, p).wait()
            st_out(g, k, p).start()
        if g + 1 < ST_GROUPS:
            for k in range(NB_S):
                st_in(g + 1, k, 1 - p).start()
        for k in range(NB_S):
            st_out(g, k, p).wait()

    row_s = pltpu.make_async_copy(state_v, sb_out.at[pl.ds(pos, 1)],
                                  sem_rows.at[0])
    row_s.start()

    it_in.wait()
    r = pos // 128
    c = pos - r * 128
    row_ids = jax.lax.broadcasted_iota(jnp.int32, (ITER_R, 128), 0)
    col_ids = jax.lax.broadcasted_iota(jnp.int32, (ITER_R, 128), 1)
    it_buf[...] = jnp.where((row_ids == r) & (col_ids == c), cnt, it_buf[...])
    it_wr.start()

    it_wr.wait()
    row_c.wait()
    row_s.wait()


def kernel(state_buffer, cam_data_buffer, iter_buffer, position, state,
           cam_data, count):
    pos2 = position.reshape(1)
    cnt2 = count.reshape(1)
    state_row = state.reshape(1, X_DIM)
    cam_row = cam_data.reshape(1, Y_FLAT)
    cam2d = cam_data_buffer.reshape(CAP, Y_FLAT)
    iter2d = iter_buffer.reshape(ITER_R, 128)

    out_sb, out_cb, out_it = pl.pallas_call(
        _push_body,
        in_specs=[
            pl.BlockSpec(memory_space=pltpu.SMEM),
            pl.BlockSpec(memory_space=pltpu.SMEM),
            pl.BlockSpec(memory_space=pltpu.VMEM),
            pl.BlockSpec(memory_space=pltpu.VMEM),
            pl.BlockSpec(memory_space=pl.ANY),
            pl.BlockSpec(memory_space=pl.ANY),
            pl.BlockSpec(memory_space=pl.ANY),
        ],
        out_specs=[
            pl.BlockSpec(memory_space=pl.ANY),
            pl.BlockSpec(memory_space=pl.ANY),
            pl.BlockSpec(memory_space=pl.ANY),
        ],
        out_shape=[
            jax.ShapeDtypeStruct((CAP, X_DIM), jnp.float32),
            jax.ShapeDtypeStruct((CAP, Y_FLAT), jnp.float32),
            jax.ShapeDtypeStruct((ITER_R, 128), jnp.int32),
        ],
        scratch_shapes=[
            pltpu.VMEM((2, NB_C, CAM_SR, Y_FLAT), jnp.float32),
            pltpu.VMEM((2, NB_S, ST_SR, X_DIM), jnp.float32),
            pltpu.VMEM((ITER_R, 128), jnp.int32),
            pltpu.SemaphoreType.DMA((2, NB_C)),
            pltpu.SemaphoreType.DMA((2, NB_C)),
            pltpu.SemaphoreType.DMA((2, NB_S)),
            pltpu.SemaphoreType.DMA((2, NB_S)),
            pltpu.SemaphoreType.DMA,
            pltpu.SemaphoreType.DMA((2,)),
        ],
    )(pos2, cnt2, state_row, cam_row, state_buffer, cam2d, iter2d)

    new_position = jnp.remainder(position + 1, CAP)
    full_buffer = (position + 1) == CAP
    return (out_sb, out_cb.reshape(CAP, Y0, Y1), out_it.reshape(CAP),
            new_position, full_buffer)
